# baseline (speedup 1.0000x reference)
import functools

import jax
import jax.numpy as jnp
from jax import lax
from jax.experimental import pallas as pl
from jax.experimental.pallas import tpu as pltpu
from jax.experimental.pallas import tpu_sc as plsc

VOCAB = 1000000
D_MODEL = 64
BATCH = 16384
HIST = 200

NC = 2
NS = 16
NW = NC * NS

B_PER_W = BATCH // NW
GB = 4
NG = B_PER_W // GB
NPAIR = NG // 2
H_SPLITS = ((0, 128), (128, 72))

SCALE = 8.0

TBLK = 1024
TGRID = -(-VOCAB // TBLK)


def _prep_body(wt_ref, o_ref):
    o_ref[:, 0:D_MODEL] = wt_ref[...].T * SCALE


def _prep_table(emb_weight):
    wt = emb_weight.T
    return pl.pallas_call(
        _prep_body,
        out_shape=jax.ShapeDtypeStruct((VOCAB, 2 * D_MODEL), jnp.float32),
        grid=(TGRID,),
        in_specs=[pl.BlockSpec((D_MODEL, TBLK), lambda i: (0, i))],
        out_specs=pl.BlockSpec((TBLK, 2 * D_MODEL), lambda i: (i, 0)),
    )(wt)


def _gather_body(table_hbm, x_hbm, out_hbm, idx0, idx1, rows0, rows1,
                 sem_g, sem_o, sem_i):
    wid = lax.axis_index("s") * NC + lax.axis_index("c")
    b0 = wid * B_PER_W
    idxb = (idx0, idx1)
    rowsb = (rows0, rows1)

    def slab(g):
        return pl.ds(b0 + g * GB, GB)

    def out_rect(g):
        return out_hbm.at[slab(g), :, pl.ds(0, D_MODEL)]

    def fire_gathers(g, p):
        for b in range(GB):
            for h0, hn in H_SPLITS:
                pltpu.async_copy(
                    table_hbm.at[idxb[p].at[b, pl.ds(h0, hn)]],
                    rowsb[p].at[b, pl.ds(h0, hn)],
                    sem_g,
                )

    def wait_gathers(p):
        for b in range(GB):
            for h0, hn in H_SPLITS:
                pltpu.make_async_copy(
                    table_hbm.at[idxb[p].at[b, pl.ds(h0, hn)]],
                    rowsb[p].at[b, pl.ds(h0, hn)],
                    sem_g,
                ).wait()

    def fire_idx(g, p):
        pltpu.async_copy(x_hbm.at[slab(g)], idxb[p], sem_i)

    def wait_idx(g, p):
        pltpu.make_async_copy(x_hbm.at[slab(g)], idxb[p], sem_i).wait()

    def fire_out(g, p):
        pltpu.async_copy(rowsb[p], out_rect(g), sem_o)

    def wait_out(g, p):
        pltpu.make_async_copy(rowsb[p], out_rect(g), sem_o).wait()

    pltpu.sync_copy(x_hbm.at[slab(0)], idxb[0])
    fire_gathers(0, 0)
    fire_idx(1, 1)

    def step(g, p):
        @pl.when(g >= 1)
        def _():
            wait_out(g - 1, 1 - p)

        @pl.when(g + 1 < NG)
        def _():
            wait_idx(g + 1, 1 - p)
        wait_gathers(p)

        @pl.when(g + 1 < NG)
        def _():
            fire_gathers(g + 1, 1 - p)
        fire_out(g, p)

        @pl.when(g + 2 < NG)
        def _():
            fire_idx(g + 2, p)

    def pair(t, _):
        g = t * 2
        step(g, 0)
        step(g + 1, 1)
        return 0

    lax.fori_loop(0, NPAIR, pair, 0)
    wait_out(NG - 1, 1)


@functools.lru_cache(maxsize=1)
def _sc_gather():
    return pl.kernel(
        _gather_body,
        out_type=jax.ShapeDtypeStruct((BATCH, HIST, 2 * D_MODEL), jnp.float32),
        mesh=plsc.VectorSubcoreMesh(
            core_axis_name="c", subcore_axis_name="s",
            num_cores=NC, num_subcores=NS,
        ),
        scratch_types=[
            pltpu.VMEM((GB, HIST), jnp.int32),
            pltpu.VMEM((GB, HIST), jnp.int32),
            pltpu.VMEM((GB, HIST, D_MODEL), jnp.float32),
            pltpu.VMEM((GB, HIST, D_MODEL), jnp.float32),
            pltpu.SemaphoreType.DMA,
            pltpu.SemaphoreType.DMA,
            pltpu.SemaphoreType.DMA,
        ],
        compiler_params=pltpu.CompilerParams(use_tc_tiling_on_sc=False),
    )


def kernel(x, emb_weight):
    w2 = _prep_table(emb_weight).reshape(2 * VOCAB, D_MODEL)
    x2 = x.astype(jnp.int32) * 2
    padded = _sc_gather()(w2, x2)
    return padded[:, :, :D_MODEL]


# speedup vs baseline: 1.1102x; 1.1102x over previous
import functools

import jax
import jax.numpy as jnp
from jax import lax
from jax.experimental import pallas as pl
from jax.experimental.pallas import tpu as pltpu
from jax.experimental.pallas import tpu_sc as plsc

VOCAB = 1000000
D_MODEL = 64
BATCH = 16384
HIST = 200

NC = 2
NS = 16
NW = NC * NS

B_PER_W = BATCH // NW
GB = 2
NG = B_PER_W // GB
NTRI = NG // 3
H_SPLITS = ((0, 128), (128, 72))

SCALE = 8.0
VECS_PER_ROW = D_MODEL // 16


def _gather_body(table_hbm, x_hbm, out_hbm, idx0, idx1, idx2,
                 rows0, rows1, rows2, sem_g, sem_o, sem_i):
    wid = lax.axis_index("s") * NC + lax.axis_index("c")
    b0 = wid * B_PER_W
    idxb = (idx0, idx1, idx2)
    rowsb = (rows0, rows1, rows2)

    def slab(g):
        return pl.ds(b0 + g * GB, GB)

    def out_rect(g):
        return out_hbm.at[slab(g), :, pl.ds(0, D_MODEL)]

    def fire_gathers(g, p):
        for b in range(GB):
            for h0, hn in H_SPLITS:
                pltpu.async_copy(
                    table_hbm.at[idxb[p].at[b, pl.ds(h0, hn)]],
                    rowsb[p].at[b, pl.ds(h0, hn)],
                    sem_g,
                )

    def wait_gathers(p):
        for b in range(GB):
            for h0, hn in H_SPLITS:
                pltpu.make_async_copy(
                    table_hbm.at[idxb[p].at[b, pl.ds(h0, hn)]],
                    rowsb[p].at[b, pl.ds(h0, hn)],
                    sem_g,
                ).wait()

    def fire_idx(g, p):
        pltpu.async_copy(x_hbm.at[slab(g)], idxb[p], sem_i)

    def wait_idx(g, p):
        pltpu.make_async_copy(x_hbm.at[slab(g)], idxb[p], sem_i).wait()

    def fire_out(g, p):
        pltpu.async_copy(rowsb[p], out_rect(g), sem_o)

    def wait_out(g, p):
        pltpu.make_async_copy(rowsb[p], out_rect(g), sem_o).wait()

    def scale_rows(p):
        buf = rowsb[p]

        def sbody(i, _):
            h = i * 4
            for dh in range(4):
                for b in range(GB):
                    for k in range(VECS_PER_ROW):
                        sl = pl.ds(k * 16, 16)
                        buf[b, h + dh, sl] = buf[b, h + dh, sl] * SCALE
            return 0

        lax.fori_loop(0, HIST // 4, sbody, 0)

    pltpu.sync_copy(x_hbm.at[slab(0)], idxb[0])
    fire_gathers(0, 0)
    fire_idx(1, 1)

    def step(g, p):
        pn = (p + 1) % 3
        pp = (p + 2) % 3

        @pl.when(g >= 2)
        def _():
            wait_out(g - 2, pn)

        @pl.when(g + 1 < NG)
        def _():
            wait_idx(g + 1, pn)
        wait_gathers(p)

        @pl.when(g + 1 < NG)
        def _():
            fire_gathers(g + 1, pn)
        scale_rows(p)
        fire_out(g, p)

        @pl.when(g + 2 < NG)
        def _():
            fire_idx(g + 2, pp)

    def tri(t, _):
        g = t * 3
        step(g, 0)
        step(g + 1, 1)
        step(g + 2, 2)
        return 0

    lax.fori_loop(0, NTRI, tri, 0)
    step(NG - 1, (NG - 1) % 3)
    wait_out(NG - 2, (NG - 2) % 3)
    wait_out(NG - 1, (NG - 1) % 3)


@functools.lru_cache(maxsize=1)
def _sc_gather():
    return pl.kernel(
        _gather_body,
        out_type=jax.ShapeDtypeStruct((BATCH, HIST, 2 * D_MODEL), jnp.float32),
        mesh=plsc.VectorSubcoreMesh(
            core_axis_name="c", subcore_axis_name="s",
            num_cores=NC, num_subcores=NS,
        ),
        scratch_types=[
            pltpu.VMEM((GB, HIST), jnp.int32),
            pltpu.VMEM((GB, HIST), jnp.int32),
            pltpu.VMEM((GB, HIST), jnp.int32),
            pltpu.VMEM((GB, HIST, D_MODEL), jnp.float32),
            pltpu.VMEM((GB, HIST, D_MODEL), jnp.float32),
            pltpu.VMEM((GB, HIST, D_MODEL), jnp.float32),
            pltpu.SemaphoreType.DMA,
            pltpu.SemaphoreType.DMA,
            pltpu.SemaphoreType.DMA,
        ],
        compiler_params=pltpu.CompilerParams(use_tc_tiling_on_sc=False),
    )


def kernel(x, emb_weight):
    wpad = jnp.pad(emb_weight, ((0, 0), (0, D_MODEL)))
    wpad = wpad.reshape(2 * VOCAB, D_MODEL)
    x2 = x.astype(jnp.int32) * 2
    padded = _sc_gather()(wpad, x2)
    return padded[:, :, :D_MODEL]
